# baseline (speedup 1.0000x reference)
import functools

import jax
import jax.numpy as jnp
from jax.experimental import pallas as pl
from jax.experimental.pallas import tpu as pltpu

_TOP_K = 2
_TS = 512
_TH = 2048


def _gate_kernel(x_ref, wg1_ref, bg1_ref, wg2_ref, bg2_ref, w_out, i_out,
                 xbf_out, acc_ref, *, n_tiles, inv_s):
    si = pl.program_id(0)
    e = wg2_ref.shape[-1]
    xb = x_ref[...]
    xbf_out[...] = xb.astype(jnp.bfloat16)
    part = jnp.sum(xb, axis=1)

    @pl.when(si == 0)
    def _():
        acc_ref[...] = part

    @pl.when(si != 0)
    def _():
        acc_ref[...] = acc_ref[...] + part

    @pl.when(si == n_tiles - 1)
    def _():
        xm = acc_ref[...] * inv_s
        gh = jnp.dot(xm, wg1_ref[...], preferred_element_type=jnp.float32,
                     precision=jax.lax.Precision.HIGHEST) + bg1_ref[...]
        gh = gh * jax.lax.logistic(gh)
        logits = jnp.dot(gh, wg2_ref[...], preferred_element_type=jnp.float32,
                         precision=jax.lax.Precision.HIGHEST) + bg2_ref[...]
        cols = jax.lax.broadcasted_iota(jnp.int32, logits.shape, 1)
        m1 = jnp.max(logits, axis=-1, keepdims=True)
        i1 = jnp.min(jnp.where(logits == m1, cols, e), axis=-1, keepdims=True)
        masked = jnp.where(cols == i1, -jnp.inf, logits)
        m2 = jnp.max(masked, axis=-1, keepdims=True)
        i2 = jnp.min(jnp.where(masked == m2, cols, e), axis=-1, keepdims=True)
        e2 = jnp.exp(m2 - m1)
        w1 = 1.0 / (1.0 + e2)
        w_out[...] = jnp.concatenate([w1, w1 * e2], axis=-1)
        i_out[...] = jnp.concatenate([i1, i2], axis=-1).astype(jnp.int32)


def _gate(x, Wg1, bg1, Wg2, bg2, *, top_k, interpret=False):
    b, s, d = x.shape
    e = Wg2.shape[1]
    n_tiles = 4 if s % 4 == 0 else 1
    ts = s // n_tiles
    grid_spec = pltpu.PrefetchScalarGridSpec(
        num_scalar_prefetch=0,
        grid=(n_tiles,),
        in_specs=[
            pl.BlockSpec((b, ts, d), lambda si: (0, si, 0)),
            pl.BlockSpec((d, d), lambda si: (0, 0)),
            pl.BlockSpec((1, d), lambda si: (0, 0)),
            pl.BlockSpec((d, e), lambda si: (0, 0)),
            pl.BlockSpec((1, e), lambda si: (0, 0)),
        ],
        out_specs=(
            pl.BlockSpec((b, top_k), lambda si: (0, 0)),
            pl.BlockSpec((b, top_k), lambda si: (0, 0)),
            pl.BlockSpec((b, ts, d), lambda si: (0, si, 0)),
        ),
        scratch_shapes=[pltpu.VMEM((b, d), jnp.float32)],
    )
    return pl.pallas_call(
        functools.partial(_gate_kernel, n_tiles=n_tiles, inv_s=1.0 / s),
        grid_spec=grid_spec,
        out_shape=(jax.ShapeDtypeStruct((b, top_k), jnp.float32),
                   jax.ShapeDtypeStruct((b, top_k), jnp.int32),
                   jax.ShapeDtypeStruct((b, s, d), jnp.bfloat16)),
        interpret=interpret,
    )(x, Wg1, bg1[None, :], Wg2, bg2[None, :])


def _moe_kernel(idx_ref, wts_ref, x_ref, w1_ref, b1_ref, w2_ref, b2_ref,
                out_ref, *, top_k):
    bk = pl.program_id(0)
    si = pl.program_id(1)
    w = wts_ref[bk]
    w1b = w1_ref[0].astype(jnp.bfloat16)
    w2b = (w * w2_ref[0]).astype(jnp.bfloat16)
    ts = x_ref.shape[1]
    half = ts // 2
    first = bk % top_k == 0

    for i in range(2):
        rows_in = pl.ds(i * half, half)
        rows_out = pl.ds(si * ts + i * half, half)
        hmat = jnp.dot(x_ref[0, rows_in, :], w1b,
                       preferred_element_type=jnp.float32) + b1_ref[0]
        hmat = hmat * jax.lax.logistic(hmat)
        contrib = jnp.dot(hmat.astype(jnp.bfloat16), w2b,
                          preferred_element_type=jnp.float32) + w * b2_ref[0]

        @pl.when(first)
        def _(contrib=contrib, rows_out=rows_out):
            out_ref[0, rows_out, :] = contrib

        @pl.when(jnp.logical_not(first))
        def _(contrib=contrib, rows_out=rows_out):
            out_ref[0, rows_out, :] = out_ref[0, rows_out, :] + contrib


def _moe(x_bf, W1, b1, W2, b2, idx_flat, wts_flat, *, ts, th, top_k,
         interpret=False):
    b, s, d = x_bf.shape
    _, _, hdim = W1.shape
    del th
    grid = (b * top_k, s // ts)
    grid_spec = pltpu.PrefetchScalarGridSpec(
        num_scalar_prefetch=2,
        grid=grid,
        in_specs=[
            pl.BlockSpec((1, ts, d), lambda bk, si, idx, wts: (bk // top_k, si, 0)),
            pl.BlockSpec((1, d, hdim), lambda bk, si, idx, wts: (idx[bk], 0, 0)),
            pl.BlockSpec((1, 1, hdim), lambda bk, si, idx, wts: (idx[bk], 0, 0)),
            pl.BlockSpec((1, hdim, d), lambda bk, si, idx, wts: (idx[bk], 0, 0)),
            pl.BlockSpec((1, 1, d), lambda bk, si, idx, wts: (idx[bk], 0, 0)),
        ],
        out_specs=pl.BlockSpec((1, s, d),
                               lambda bk, si, idx, wts: (bk // top_k, 0, 0)),
    )
    return pl.pallas_call(
        functools.partial(_moe_kernel, top_k=top_k),
        grid_spec=grid_spec,
        out_shape=jax.ShapeDtypeStruct((b, s, d), jnp.float32),
        compiler_params=pltpu.CompilerParams(vmem_limit_bytes=63 * 1024 * 1024),
        interpret=interpret,
    )(idx_flat, wts_flat, x_bf, W1, b1[:, None, :], W2, b2[:, None, :])


def kernel(x, Wg1, bg1, Wg2, bg2, W1, b1, W2, b2):
    wts, idx, x_bf = _gate(x, Wg1, bg1, Wg2, bg2, top_k=_TOP_K)
    out = _moe(x_bf, W1, b1, W2, b2, idx.reshape(-1), wts.reshape(-1),
               ts=_TS, th=_TH, top_k=_TOP_K)
    return (out, (wts, idx))


# speedup vs baseline: 1.1038x; 1.1038x over previous
import functools

import jax
import jax.numpy as jnp
from jax.experimental import pallas as pl
from jax.experimental.pallas import tpu as pltpu

_TOP_K = 2
_TS = 1024
_TH = 2048


def _gate_kernel(x_ref, wg1_ref, bg1_ref, wg2_ref, bg2_ref, w_out, i_out,
                 xbf_out, acc_ref, *, n_tiles, inv_s):
    si = pl.program_id(0)
    e = wg2_ref.shape[-1]
    xb = x_ref[...]
    xbf_out[...] = xb.astype(jnp.bfloat16)
    part = jnp.sum(xb, axis=1)

    @pl.when(si == 0)
    def _():
        acc_ref[...] = part

    @pl.when(si != 0)
    def _():
        acc_ref[...] = acc_ref[...] + part

    @pl.when(si == n_tiles - 1)
    def _():
        xm = acc_ref[...] * inv_s
        gh = jnp.dot(xm, wg1_ref[...], preferred_element_type=jnp.float32,
                     precision=jax.lax.Precision.HIGHEST) + bg1_ref[...]
        gh = gh * jax.lax.logistic(gh)
        logits = jnp.dot(gh, wg2_ref[...], preferred_element_type=jnp.float32,
                         precision=jax.lax.Precision.HIGHEST) + bg2_ref[...]
        cols = jax.lax.broadcasted_iota(jnp.int32, logits.shape, 1)
        m1 = jnp.max(logits, axis=-1, keepdims=True)
        i1 = jnp.min(jnp.where(logits == m1, cols, e), axis=-1, keepdims=True)
        masked = jnp.where(cols == i1, -jnp.inf, logits)
        m2 = jnp.max(masked, axis=-1, keepdims=True)
        i2 = jnp.min(jnp.where(masked == m2, cols, e), axis=-1, keepdims=True)
        e2 = jnp.exp(m2 - m1)
        w1 = 1.0 / (1.0 + e2)
        w_out[...] = jnp.concatenate([w1, w1 * e2], axis=-1)
        i_out[...] = jnp.concatenate([i1, i2], axis=-1).astype(jnp.int32)


def _gate(x, Wg1, bg1, Wg2, bg2, *, top_k, interpret=False):
    b, s, d = x.shape
    e = Wg2.shape[1]
    n_tiles = 4 if s % 4 == 0 else 1
    ts = s // n_tiles
    grid_spec = pltpu.PrefetchScalarGridSpec(
        num_scalar_prefetch=0,
        grid=(n_tiles,),
        in_specs=[
            pl.BlockSpec((b, ts, d), lambda si: (0, si, 0)),
            pl.BlockSpec((d, d), lambda si: (0, 0)),
            pl.BlockSpec((1, d), lambda si: (0, 0)),
            pl.BlockSpec((d, e), lambda si: (0, 0)),
            pl.BlockSpec((1, e), lambda si: (0, 0)),
        ],
        out_specs=(
            pl.BlockSpec((b, top_k), lambda si: (0, 0)),
            pl.BlockSpec((b, top_k), lambda si: (0, 0)),
            pl.BlockSpec((b, ts, d), lambda si: (0, si, 0)),
        ),
        scratch_shapes=[pltpu.VMEM((b, d), jnp.float32)],
    )
    return pl.pallas_call(
        functools.partial(_gate_kernel, n_tiles=n_tiles, inv_s=1.0 / s),
        grid_spec=grid_spec,
        out_shape=(jax.ShapeDtypeStruct((b, top_k), jnp.float32),
                   jax.ShapeDtypeStruct((b, top_k), jnp.int32),
                   jax.ShapeDtypeStruct((b, s, d), jnp.bfloat16)),
        interpret=interpret,
    )(x, Wg1, bg1[None, :], Wg2, bg2[None, :])


def _moe_kernel(idx_ref, wts_ref, x_ref, w1_ref, b1_ref, w2_ref, b2_ref,
                out_ref, *, top_k):
    bk = pl.program_id(1)
    w = wts_ref[bk]
    w1b = w1_ref[0].astype(jnp.bfloat16)
    w2b = (w * w2_ref[0]).astype(jnp.bfloat16)
    ts = x_ref.shape[1]
    half = ts // 2
    first = bk % top_k == 0

    for i in range(2):
        rows = pl.ds(i * half, half)
        hmat = jnp.dot(x_ref[0, rows, :], w1b,
                       preferred_element_type=jnp.float32) + b1_ref[0]
        hmat = hmat * jax.lax.logistic(hmat)
        contrib = jnp.dot(hmat.astype(jnp.bfloat16), w2b,
                          preferred_element_type=jnp.float32) + w * b2_ref[0]

        @pl.when(first)
        def _(contrib=contrib, rows=rows):
            out_ref[0, rows, :] = contrib

        @pl.when(jnp.logical_not(first))
        def _(contrib=contrib, rows=rows):
            out_ref[0, rows, :] = out_ref[0, rows, :] + contrib


def _moe(x_bf, W1, b1, W2, b2, idx_flat, wts_flat, *, ts, th, top_k,
         interpret=False):
    b, s, d = x_bf.shape
    _, _, hdim = W1.shape
    del th
    grid = (s // ts, b * top_k)
    grid_spec = pltpu.PrefetchScalarGridSpec(
        num_scalar_prefetch=2,
        grid=grid,
        in_specs=[
            pl.BlockSpec((1, ts, d), lambda si, bk, idx, wts: (bk // top_k, si, 0)),
            pl.BlockSpec((1, d, hdim), lambda si, bk, idx, wts: (idx[bk], 0, 0)),
            pl.BlockSpec((1, 1, hdim), lambda si, bk, idx, wts: (idx[bk], 0, 0)),
            pl.BlockSpec((1, hdim, d), lambda si, bk, idx, wts: (idx[bk], 0, 0)),
            pl.BlockSpec((1, 1, d), lambda si, bk, idx, wts: (idx[bk], 0, 0)),
        ],
        out_specs=pl.BlockSpec((1, ts, d),
                               lambda si, bk, idx, wts: (bk // top_k, si, 0)),
    )
    return pl.pallas_call(
        functools.partial(_moe_kernel, top_k=top_k),
        grid_spec=grid_spec,
        out_shape=jax.ShapeDtypeStruct((b, s, d), jnp.float32),
        compiler_params=pltpu.CompilerParams(vmem_limit_bytes=63 * 1024 * 1024),
        interpret=interpret,
    )(idx_flat, wts_flat, x_bf, W1, b1[:, None, :], W2, b2[:, None, :])


def kernel(x, Wg1, bg1, Wg2, bg2, W1, b1, W2, b2):
    wts, idx, x_bf = _gate(x, Wg1, bg1, Wg2, bg2, top_k=_TOP_K)
    out = _moe(x_bf, W1, b1, W2, b2, idx.reshape(-1), wts.reshape(-1),
               ts=_TS, th=_TH, top_k=_TOP_K)
    return (out, (wts, idx))
